# Initial kernel scaffold; baseline (speedup 1.0000x reference)
import jax
import jax.numpy as jnp
from jax.experimental import pallas as pl


def kernel(input, embed):
    raise NotImplementedError("write your pallas kernel here")


# speedup vs baseline: 1.7712x; 1.7712x over previous
import jax
import jax.numpy as jnp
from jax.experimental import pallas as pl

_NUM_TOKENS = 1024
_DIM = 256
_B = 8
_HW = 1024


def _vq_body(x_ref, e_ref, q_ref, ind_ref, diff_ref):
    x = x_ref[...]
    e = e_ref[...]
    scores = jnp.dot(x, e, preferred_element_type=jnp.float32)
    x2 = jnp.sum(x * x, axis=1, keepdims=True)
    e2 = jnp.sum(e * e, axis=0, keepdims=True)
    neg = -(x2 - 2.0 * scores + e2)
    m = jnp.max(neg, axis=1)
    iota_c = jax.lax.broadcasted_iota(jnp.int32, (_HW, _NUM_TOKENS), 1)
    idx = jnp.min(jnp.where(neg == m[:, None], iota_c, _NUM_TOKENS), axis=1)
    iota_r = jax.lax.broadcasted_iota(jnp.int32, (_NUM_TOKENS, _HW), 0)
    onehot_t = (iota_r == idx[None, :]).astype(jnp.float32)
    q_ref[0] = jnp.dot(e, onehot_t, preferred_element_type=jnp.float32)
    ind_ref[0, 0] = idx
    diff_ref[0] = -m.reshape(_HW // 128, 128)


def kernel(input, embed):
    flat = input.reshape(_B * _HW, _DIM)
    qT, ind, diffp = pl.pallas_call(
        _vq_body,
        grid=(_B,),
        in_specs=[
            pl.BlockSpec((_HW, _DIM), lambda i: (i, 0)),
            pl.BlockSpec((_DIM, _NUM_TOKENS), lambda i: (0, 0)),
        ],
        out_specs=[
            pl.BlockSpec((1, _DIM, _HW), lambda i: (i, 0, 0)),
            pl.BlockSpec((1, 1, _HW), lambda i: (i, 0, 0)),
            pl.BlockSpec((1, _HW // 128, 128), lambda i: (i, 0, 0)),
        ],
        out_shape=[
            jax.ShapeDtypeStruct((_B, _DIM, _HW), jnp.float32),
            jax.ShapeDtypeStruct((_B, 1, _HW), jnp.int32),
            jax.ShapeDtypeStruct((_B, _HW // 128, 128), jnp.float32),
        ],
    )(flat, embed)
    quantize = qT.reshape(_B, _DIM, 32, 32)
    embed_ind = ind.reshape(_B, 32, 32)
    diff = jnp.sum(diffp) / (_B * _HW * _DIM)
    return (quantize, diff, embed_ind)


# speedup vs baseline: 1.8652x; 1.0531x over previous
import jax
import jax.numpy as jnp
from jax.experimental import pallas as pl

_NUM_TOKENS = 1024
_DIM = 256
_B = 8
_HW = 1024


def _vq_body(x_ref, e_ref, q_ref, ind_ref, diff_ref):
    x = x_ref[...]
    e = e_ref[...]
    scores = jnp.dot(x, e, preferred_element_type=jnp.float32)
    x2 = jnp.sum(x * x, axis=1, keepdims=True)
    e2 = jnp.sum(e * e, axis=0, keepdims=True)
    dist = (x2 - 2.0 * scores) + e2
    dist_t = dist.T
    md = jnp.min(dist_t, axis=0, keepdims=True)
    iota_s = jax.lax.broadcasted_iota(jnp.int32, (_NUM_TOKENS, _HW), 0)
    idx = jnp.min(jnp.where(dist_t == md, iota_s, _NUM_TOKENS),
                  axis=0, keepdims=True)
    onehot_t = (iota_s == idx).astype(jnp.float32)
    q_ref[0] = jnp.dot(e, onehot_t, preferred_element_type=jnp.float32)
    ind_ref[0] = idx
    diff_ref[0] = md


def kernel(input, embed):
    flat = input.reshape(_B * _HW, _DIM)
    qT, ind, diffp = pl.pallas_call(
        _vq_body,
        grid=(_B,),
        in_specs=[
            pl.BlockSpec((_HW, _DIM), lambda i: (i, 0)),
            pl.BlockSpec((_DIM, _NUM_TOKENS), lambda i: (0, 0)),
        ],
        out_specs=[
            pl.BlockSpec((1, _DIM, _HW), lambda i: (i, 0, 0)),
            pl.BlockSpec((1, 1, _HW), lambda i: (i, 0, 0)),
            pl.BlockSpec((1, 1, _HW), lambda i: (i, 0, 0)),
        ],
        out_shape=[
            jax.ShapeDtypeStruct((_B, _DIM, _HW), jnp.float32),
            jax.ShapeDtypeStruct((_B, 1, _HW), jnp.int32),
            jax.ShapeDtypeStruct((_B, 1, _HW), jnp.float32),
        ],
    )(flat, embed)
    quantize = qT.reshape(_B, _DIM, 32, 32)
    embed_ind = ind.reshape(_B, 32, 32)
    diff = jnp.sum(diffp) / (_B * _HW * _DIM)
    return (quantize, diff, embed_ind)
